# baseline (speedup 1.0000x reference)
import functools

import jax
import jax.numpy as jnp
from jax import lax
from jax.experimental import pallas as pl
from jax.experimental.pallas import tpu as pltpu
from jax.experimental.pallas import tpu_sc as plsc

NUM_CORES = 2
NUM_WORKERS = 32

T_DIM = 200
S_DIM = 4096
S_BLK = 128
C_DIM = S_DIM // S_BLK
D = 64
UNITS = T_DIM * C_DIM
U_PER_W = UNITS // NUM_WORKERS

V = 1000000
VBLK_FULL = V // S_BLK
V_TAIL = V - VBLK_FULL * S_BLK
NSLOT = 4
A_ITERS = VBLK_FULL // (NSLOT * NUM_WORKERS)


def _mesh():
    return plsc.VectorSubcoreMesh(core_axis_name="c", subcore_axis_name="s")


@jax.jit
def _prep(table_t, tail_t):
    @functools.partial(
        pl.kernel,
        mesh=_mesh(),
        out_type=jax.ShapeDtypeStruct((V // 2, 2 * D), jnp.float32),
        compiler_params=pltpu.CompilerParams(needs_layout_passes=False),
        scratch_types=[
            *[pltpu.VMEM((D, S_BLK), jnp.float32) for _ in range(NSLOT)],
            *[pltpu.VMEM((S_BLK // 2, 2 * D), jnp.float32) for _ in range(NSLOT)],
            *[pltpu.SemaphoreType.DMA for _ in range(2 * NSLOT)],
        ],
    )
    def ka(tt_hbm, tail_hbm, tp_hbm, *scr):
        chs = scr[:NSLOT]
        tbs = scr[NSLOT : 2 * NSLOT]
        gsem = scr[2 * NSLOT : 3 * NSLOT]
        ssem = scr[3 * NSLOT :]
        wid = lax.axis_index("s") * NUM_CORES + lax.axis_index("c")
        iota = lax.iota(jnp.int32, 16)
        ridxs = [iota + sl0 * 16 for sl0 in range(S_BLK // 16)]
        def diag_ref(k):
            return (iota + k) & 15

        pair_rows = [r >> 1 for r in ridxs]
        cbase = [(r & 1) * D for r in ridxs]

        def blk_of(i, b):
            return (NSLOT * i + b) * NUM_WORKERS + wid

        def load_chunk(blk, b):
            off = pl.multiple_of(blk * S_BLK, S_BLK)
            pltpu.async_copy(
                tt_hbm.at[:, pl.ds(off, S_BLK)], chs[b], gsem[b]
            )

        def wait_chunk(b):
            pltpu.make_async_copy(
                tt_hbm.at[:, pl.ds(0, S_BLK)], chs[b], gsem[b]
            ).wait()

        def fire_store(blk, b):
            off = pl.multiple_of(blk * (S_BLK // 2), S_BLK // 2)
            pltpu.async_copy(
                tbs[b], tp_hbm.at[pl.ds(off, S_BLK // 2), :], ssem[b]
            )

        def wait_store(b):
            pltpu.make_async_copy(
                tbs[b], tp_hbm.at[pl.ds(0, S_BLK // 2), :], ssem[b]
            ).wait()

        def transpose(b):
            @plsc.parallel_loop(0, 16, unroll=4)
            def _tr(k):
                dg = diag_ref(k)
                for d0 in range(0, D, 16):
                    rowv = dg + d0
                    for sl0 in range(S_BLK // 16):
                        v = plsc.load_gather(chs[b], [rowv, ridxs[sl0]])
                        plsc.store_scatter(
                            tbs[b], [pair_rows[sl0], cbase[sl0] + rowv], v
                        )

        for b in range(NSLOT):
            load_chunk(blk_of(0, b), b)

        def body(i, carry):
            for b in range(NSLOT):
                wait_chunk(b)

                @pl.when(i > 0)
                def _():
                    wait_store(b)

                transpose(b)
                fire_store(blk_of(i, b), b)

                @pl.when(i + 1 < A_ITERS)
                def _():
                    load_chunk(blk_of(i + 1, b), b)

            return carry

        lax.fori_loop(0, A_ITERS, body, 0)
        for b in range(NSLOT):
            wait_store(b)

        left0 = NSLOT * A_ITERS * NUM_WORKERS

        @pl.when(wid < VBLK_FULL - left0)
        def _():
            blk = left0 + wid
            load_chunk(blk, 0)
            wait_chunk(0)
            transpose(0)
            fire_store(blk, 0)
            wait_store(0)

        @pl.when(wid == 8)
        def _():
            pltpu.async_copy(tail_hbm, chs[1], gsem[1])
            pltpu.make_async_copy(tail_hbm, chs[1], gsem[1]).wait()

            transpose(1)

            pltpu.async_copy(
                tbs[1],
                tp_hbm.at[pl.ds((V - S_BLK) // 2, S_BLK // 2), :],
                ssem[1],
            )
            pltpu.make_async_copy(
                tbs[1], tp_hbm.at[pl.ds(0, S_BLK // 2), :], ssem[1]
            ).wait()

    return ka(table_t, tail_t)


@jax.jit
def _embed(x_t_flat, tp):
    @functools.partial(
        pl.kernel,
        mesh=_mesh(),
        out_type=jax.ShapeDtypeStruct((T_DIM, 8, C_DIM, 8, S_BLK), jnp.float32),
        compiler_params=pltpu.CompilerParams(
            use_tc_tiling_on_sc=False, needs_layout_passes=False
        ),
        scratch_types=[
            *[pltpu.VMEM((S_BLK,), jnp.int32) for _ in range(NSLOT)],
            *[pltpu.VMEM((S_BLK, D), jnp.float32) for _ in range(NSLOT)],
            *[pltpu.VMEM((8, 8, S_BLK + 1), jnp.float32) for _ in range(NSLOT)],
            *[pltpu.SemaphoreType.DMA for _ in range(2 * NSLOT)],
        ],
    )
    def kb(x_hbm, tp_hbm, out_hbm, *scr):
        idxs = scr[:NSLOT]
        rows = scr[NSLOT : 2 * NSLOT]
        obs = scr[2 * NSLOT : 3 * NSLOT]
        gsem = scr[3 * NSLOT : 4 * NSLOT]
        ssem = scr[4 * NSLOT :]
        wid = lax.axis_index("s") * NUM_CORES + lax.axis_index("c")
        u_base = wid * U_PER_W
        iota = lax.iota(jnp.int32, 16)
        gconst = [(iota + d0) >> 3 for d0 in range(0, D, 16)]
        dsconst = [(iota + d0) & 7 for d0 in range(0, D, 16)]

        def load_idx(u, b):
            pltpu.sync_copy(x_hbm.at[pl.ds(u * S_BLK, S_BLK)], idxs[b])

        def fire_gather(b):
            pltpu.async_copy(tp_hbm.at[idxs[b]], rows[b], gsem[b])

        def wait_gather(b):
            pltpu.make_async_copy(tp_hbm.at[idxs[b]], rows[b], gsem[b]).wait()

        def fire_store(u, b):
            t = u // C_DIM
            c = lax.rem(u, C_DIM)
            pltpu.async_copy(
                obs[b].at[:, :, pl.ds(0, S_BLK)], out_hbm.at[t, :, c], ssem[b]
            )

        def wait_store(b):
            pltpu.make_async_copy(
                obs[b].at[:, :, pl.ds(0, S_BLK)], out_hbm.at[0, :, 0], ssem[b]
            ).wait()

        def transpose(b):
            @plsc.parallel_loop(0, S_BLK, unroll=4)
            def _tr(sl):
                slv = jnp.full((16,), sl, jnp.int32)
                for q in range(D // 16):
                    v = rows[b][sl, pl.ds(q * 16, 16)]
                    plsc.store_scatter(obs[b], [gconst[q], dsconst[q], slv], v)

        for b in range(NSLOT):
            load_idx(u_base + b, b)
            fire_gather(b)

        def body(i, carry):
            for b in range(NSLOT):
                j = NSLOT * i + b
                u = u_base + j
                wait_gather(b)

                @pl.when(i > 0)
                def _():
                    wait_store(b)

                transpose(b)
                fire_store(u, b)

                @pl.when(j + NSLOT < U_PER_W)
                def _():
                    load_idx(u + NSLOT, b)
                    fire_gather(b)

            return carry

        lax.fori_loop(0, U_PER_W // NSLOT, body, 0)
        for b in range(NSLOT):
            wait_store(b)

    return kb(x_t_flat, tp)


def kernel(x, table):
    xf = x.T.reshape(-1)
    tp = _prep(table.T, table[V - S_BLK :, :].T)
    out5 = _embed(xf, tp.reshape(V, D))
    return out5.transpose(2, 4, 0, 1, 3).reshape(S_DIM, T_DIM, D)


# speedup vs baseline: 1.0565x; 1.0565x over previous
import functools

import jax
import jax.numpy as jnp
from jax import lax
from jax.experimental import pallas as pl
from jax.experimental.pallas import tpu as pltpu
from jax.experimental.pallas import tpu_sc as plsc

NUM_CORES = 2
NUM_WORKERS = 32

T_DIM = 200
S_DIM = 4096
S_BLK = 128
C_DIM = S_DIM // S_BLK
D = 64
UNITS = T_DIM * C_DIM
U_PER_W = UNITS // NUM_WORKERS

V = 1000000
VBLK_FULL = V // S_BLK
V_TAIL = V - VBLK_FULL * S_BLK
A_ITERS = VBLK_FULL // (2 * NUM_WORKERS)


def _mesh():
    return plsc.VectorSubcoreMesh(core_axis_name="c", subcore_axis_name="s")


@jax.jit
def _prep(table_t, tail_t):
    @functools.partial(
        pl.kernel,
        mesh=_mesh(),
        out_type=jax.ShapeDtypeStruct((V // 2, 2 * D), jnp.float32),
        compiler_params=pltpu.CompilerParams(needs_layout_passes=False),
        scratch_types=[
            *[pltpu.VMEM((D, S_BLK), jnp.float32) for _ in range(2)],
            *[pltpu.VMEM((S_BLK // 2, 2 * D), jnp.float32) for _ in range(2)],
            *[pltpu.SemaphoreType.DMA for _ in range(4)],
        ],
    )
    def ka(tt_hbm, tail_hbm, tp_hbm, ch0, ch1, tb0, tb1, g0, g1, s0, s1):
        chs, tbs = (ch0, ch1), (tb0, tb1)
        gsem, ssem = (g0, g1), (s0, s1)
        wid = lax.axis_index("s") * NUM_CORES + lax.axis_index("c")
        iota = lax.iota(jnp.int32, 16)
        ridxs = [iota + sl0 * 16 for sl0 in range(S_BLK // 16)]
        def diag_ref(k):
            return (iota + k) & 15

        pair_rows = [r >> 1 for r in ridxs]
        cbase = [(r & 1) * D for r in ridxs]

        def blk_of(i, b):
            return (2 * i + b) * NUM_WORKERS + wid

        def load_chunk(blk, b):
            off = pl.multiple_of(blk * S_BLK, S_BLK)
            pltpu.async_copy(
                tt_hbm.at[:, pl.ds(off, S_BLK)], chs[b], gsem[b]
            )

        def wait_chunk(b):
            pltpu.make_async_copy(
                tt_hbm.at[:, pl.ds(0, S_BLK)], chs[b], gsem[b]
            ).wait()

        def fire_store(blk, b):
            off = pl.multiple_of(blk * (S_BLK // 2), S_BLK // 2)
            pltpu.async_copy(
                tbs[b], tp_hbm.at[pl.ds(off, S_BLK // 2), :], ssem[b]
            )

        def wait_store(b):
            pltpu.make_async_copy(
                tbs[b], tp_hbm.at[pl.ds(0, S_BLK // 2), :], ssem[b]
            ).wait()

        def transpose(b):
            @plsc.parallel_loop(0, 16, unroll=4)
            def _tr(k):
                dg = diag_ref(k)
                for d0 in range(0, D, 16):
                    rowv = dg + d0
                    for sl0 in range(S_BLK // 16):
                        v = plsc.load_gather(chs[b], [rowv, ridxs[sl0]])
                        plsc.store_scatter(
                            tbs[b], [pair_rows[sl0], cbase[sl0] + rowv], v
                        )

        for b in range(2):
            load_chunk(blk_of(0, b), b)

        def body(i, carry):
            for b in range(2):
                wait_chunk(b)

                @pl.when(i > 0)
                def _():
                    wait_store(b)

                transpose(b)
                fire_store(blk_of(i, b), b)

                @pl.when(2 * i + b + 2 < 2 * A_ITERS)
                def _():
                    load_chunk(blk_of(i + 1, b), b)

            return carry

        lax.fori_loop(0, A_ITERS, body, 0)
        for b in range(2):
            wait_store(b)

        left0 = 2 * A_ITERS * NUM_WORKERS

        @pl.when(wid < VBLK_FULL - left0)
        def _():
            blk = left0 + wid
            load_chunk(blk, 0)
            wait_chunk(0)
            transpose(0)
            fire_store(blk, 0)
            wait_store(0)

        @pl.when(wid == 8)
        def _():
            pltpu.async_copy(tail_hbm, chs[1], gsem[1])
            pltpu.make_async_copy(tail_hbm, chs[1], gsem[1]).wait()

            transpose(1)

            pltpu.async_copy(
                tbs[1],
                tp_hbm.at[pl.ds((V - S_BLK) // 2, S_BLK // 2), :],
                ssem[1],
            )
            pltpu.make_async_copy(
                tbs[1], tp_hbm.at[pl.ds(0, S_BLK // 2), :], ssem[1]
            ).wait()

    return ka(table_t, tail_t)


@jax.jit
def _embed(x_t_flat, tp):
    @functools.partial(
        pl.kernel,
        mesh=_mesh(),
        out_type=jax.ShapeDtypeStruct((T_DIM, 8, C_DIM, 8, S_BLK), jnp.float32),
        compiler_params=pltpu.CompilerParams(
            use_tc_tiling_on_sc=False, needs_layout_passes=False
        ),
        scratch_types=[
            *[pltpu.VMEM((S_BLK,), jnp.int32) for _ in range(2)],
            *[pltpu.VMEM((S_BLK, D), jnp.float32) for _ in range(2)],
            *[pltpu.VMEM((8, 8, S_BLK + 1), jnp.float32) for _ in range(2)],
            *[pltpu.SemaphoreType.DMA for _ in range(4)],
        ],
    )
    def kb(x_hbm, tp_hbm, out_hbm, idx0, idx1, rows0, rows1, ob0, ob1,
           g0, g1, s0, s1):
        idxs, rows, obs = (idx0, idx1), (rows0, rows1), (ob0, ob1)
        gsem, ssem = (g0, g1), (s0, s1)
        wid = lax.axis_index("s") * NUM_CORES + lax.axis_index("c")
        u_base = wid * U_PER_W
        iota = lax.iota(jnp.int32, 16)
        gconst = [(iota + d0) >> 3 for d0 in range(0, D, 16)]
        dsconst = [(iota + d0) & 7 for d0 in range(0, D, 16)]

        def load_idx(u, b):
            pltpu.sync_copy(x_hbm.at[pl.ds(u * S_BLK, S_BLK)], idxs[b])

        def fire_gather(b):
            pltpu.async_copy(tp_hbm.at[idxs[b]], rows[b], gsem[b])

        def wait_gather(b):
            pltpu.make_async_copy(tp_hbm.at[idxs[b]], rows[b], gsem[b]).wait()

        def fire_store(u, b):
            t = u // C_DIM
            c = lax.rem(u, C_DIM)
            pltpu.async_copy(
                obs[b].at[:, :, pl.ds(0, S_BLK)], out_hbm.at[t, :, c], ssem[b]
            )

        def wait_store(b):
            pltpu.make_async_copy(
                obs[b].at[:, :, pl.ds(0, S_BLK)], out_hbm.at[0, :, 0], ssem[b]
            ).wait()

        def transpose(b):
            @plsc.parallel_loop(0, S_BLK, unroll=4)
            def _tr(sl):
                slv = jnp.full((16,), sl, jnp.int32)
                for q in range(D // 16):
                    v = rows[b][sl, pl.ds(q * 16, 16)]
                    plsc.store_scatter(obs[b], [gconst[q], dsconst[q], slv], v)

        for b in range(2):
            load_idx(u_base + b, b)
            fire_gather(b)

        def body(i, carry):
            for b in range(2):
                j = 2 * i + b
                u = u_base + j
                wait_gather(b)

                @pl.when(i > 0)
                def _():
                    wait_store(b)

                transpose(b)
                fire_store(u, b)

                @pl.when(j + 2 < U_PER_W)
                def _():
                    load_idx(u + 2, b)
                    fire_gather(b)

            return carry

        lax.fori_loop(0, U_PER_W // 2, body, 0)
        for b in range(2):
            wait_store(b)

    return kb(x_t_flat, tp)


def kernel(x, table):
    xf = x.T.reshape(-1)
    tp = _prep(table.T, table[V - S_BLK :, :].T)
    out5 = _embed(xf, tp.reshape(V, D))
    return out5.transpose(2, 4, 0, 1, 3).reshape(S_DIM, T_DIM, D)
